# baseline (speedup 1.0000x reference)
import functools

import jax
import jax.numpy as jnp
from jax import lax
from jax.experimental import pallas as pl
from jax.experimental.pallas import tpu as pltpu
from jax.experimental.pallas import tpu_sc as plsc

NC = 2
NS = 16
L = 16
CHUNK = 48
PAGE = 24
NG = 3
NW = 2
PAGES_PER_TILE = 9


def _scale_rows(gb_ref, sb_ref, vals_ref, base, d):
  def scale_group(g, carry):
    vv = vals_ref[pl.ds(base + g * L, L)]
    for j in range(L):
      vj = lax.gather(
          vv, jnp.full((L, 1), j, jnp.int32),
          lax.GatherDimensionNumbers(offset_dims=(),
                                     collapsed_slice_dims=(0,),
                                     start_index_map=(0,)),
          slice_sizes=(1,),
          mode=lax.GatherScatterMode.PROMISE_IN_BOUNDS)
      for k in range(d // L):
        sl = (g * L + j, pl.ds(k * L, L))
        sb_ref[sl] = gb_ref[sl] * vj
    return carry

  lax.fori_loop(0, CHUNK // L, scale_group, 0, unroll=False)


def _sc_spmm_body(src_hbm, dst_hbm, vals_hbm, x_hbm, zeros_hbm, part_hbm,
                  acc_sh, src_v, dst_v, vals_v,
                  gb0, gb1, gb2, sb0, sb1,
                  g0, g1, g2, w0, w1):
  n, d = x_hbm.shape
  zrows = (n // NS) // 8 * 8
  rem = n - NS * zrows
  c = lax.axis_index("c")
  s = lax.axis_index("s")
  gbufs = (gb0, gb1, gb2)
  gsems = (g0, g1, g2)
  sbufs = (sb0, sb1)
  wsems = (w0, w1)

  pltpu.sync_copy(zeros_hbm.at[pl.ds(s * zrows, zrows)],
                  acc_sh.at[pl.ds(s * zrows, zrows)])
  if rem:
    @pl.when(s == 0)
    def _():
      pltpu.sync_copy(zeros_hbm.at[pl.ds(NS * zrows, rem)],
                      acc_sh.at[pl.ds(NS * zrows, rem)])
  plsc.subcore_barrier()

  def gather_rows(chunk, rows_ref, sem):
    idx = src_v.at[pl.ds(chunk * CHUNK, CHUNK)]
    return pltpu.async_copy(x_hbm.at[idx], rows_ref, sem)

  def wait_gather(rows_ref, sem):
    idx = src_v.at[pl.ds(0, CHUNK)]
    pltpu.make_async_copy(x_hbm.at[idx], rows_ref, sem).wait()

  def scatter_rows(chunk, rows_ref, sem):
    return pltpu.async_copy(rows_ref, acc_sh.at[dst_v.at[chunk]], sem,
                            add=True)

  def wait_scatter(rows_ref, sem):
    pltpu.make_async_copy(rows_ref, acc_sh.at[dst_v.at[0]], sem).wait()

  last = PAGE - 1

  def do_chunk(ch, j, scatter_pending):
    gi = j % NG
    si = j % NW
    wait_gather(gbufs[gi], gsems[gi])
    if j < NW:
      @pl.when(scatter_pending)
      def _():
        wait_scatter(sbufs[si], wsems[si])
    else:
      wait_scatter(sbufs[si], wsems[si])
    _scale_rows(gbufs[gi], sbufs[si], vals_v, ch * CHUNK, d)
    scatter_rows(ch, sbufs[si], wsems[si])
    gather_rows(jnp.minimum(ch + NG, last), gbufs[gi], gsems[gi])

  def do_page(page):
    pltpu.sync_copy(src_hbm.at[pl.ds(page * PAGE * CHUNK, PAGE * CHUNK)],
                    src_v)
    pltpu.sync_copy(vals_hbm.at[pl.ds(page * PAGE * CHUNK, PAGE * CHUNK)],
                    vals_v)
    pltpu.sync_copy(dst_hbm.at[pl.ds(page * PAGE, PAGE)], dst_v)

    for gi in range(NG):
      gather_rows(gi, gbufs[gi], gsems[gi])

    def hex_body(i, carry):
      base = 6 * i
      for j in range(6):
        do_chunk(base + j, j, scatter_pending=(i > 0))
      return carry

    lax.fori_loop(0, PAGE // 6, hex_body, 0, unroll=False)
    for gi in range(NG):
      wait_gather(gbufs[gi], gsems[gi])
    for si in range(NW):
      wait_scatter(sbufs[si], wsems[si])

  def page_body(p, carry):
    do_page((c * NS + s) * PAGES_PER_TILE + p)
    return carry

  lax.fori_loop(0, PAGES_PER_TILE, page_body, 0, unroll=False)
  plsc.subcore_barrier()

  pltpu.sync_copy(acc_sh.at[pl.ds(s * zrows, zrows)],
                  part_hbm.at[pl.ds(c * n + s * zrows, zrows)])
  if rem:
    @pl.when(s == 0)
    def _():
      pltpu.sync_copy(acc_sh.at[pl.ds(NS * zrows, rem)],
                      part_hbm.at[pl.ds(c * n + NS * zrows, rem)])


def _tc_finish_body(a_ref, p_ref, x_ref, w_ref, o_ref):
  a = a_ref[0]
  blended = (1.0 - a) * (p_ref[0] + p_ref[1]) + a * x_ref[...]
  o_ref[...] = jnp.dot(blended, w_ref[...], preferred_element_type=jnp.float32)


def kernel(edge_index, adj_vals, x, alpha, W):
  n, d_in = x.shape
  d_out = W.shape[1]
  e = adj_vals.shape[0]

  dst = edge_index[0]
  src = edge_index[1]
  e_pad = NC * NS * PAGES_PER_TILE * PAGE * CHUNK
  assert e_pad >= e, "page split must cover all edges"
  if e_pad != e:
    pad = e_pad - e
    spread = jnp.arange(pad, dtype=jnp.int32) % n
    src = jnp.concatenate([src, spread])
    dst = jnp.concatenate([dst, spread])
    vals = jnp.concatenate([adj_vals, jnp.zeros((pad,), adj_vals.dtype)])
  else:
    vals = adj_vals
  dst2 = dst.reshape(e_pad // CHUNK, CHUNK)
  zeros = jnp.zeros((n, d_in), jnp.float32)

  mesh = plsc.VectorSubcoreMesh(core_axis_name="c", subcore_axis_name="s")
  part = pl.kernel(
      _sc_spmm_body,
      out_type=jax.ShapeDtypeStruct((NC * n, d_in), jnp.float32),
      mesh=mesh,
      scratch_types=[
          pltpu.VMEM_SHARED((n, d_in), jnp.float32),
          pltpu.VMEM((PAGE * CHUNK,), jnp.int32),
          pltpu.VMEM((PAGE, CHUNK), jnp.int32),
          pltpu.VMEM((PAGE * CHUNK,), jnp.float32),
          pltpu.VMEM((CHUNK, d_in), jnp.float32),
          pltpu.VMEM((CHUNK, d_in), jnp.float32),
          pltpu.VMEM((CHUNK, d_in), jnp.float32),
          pltpu.VMEM((CHUNK, d_in), jnp.float32),
          pltpu.VMEM((CHUNK, d_in), jnp.float32),
          pltpu.SemaphoreType.DMA,
          pltpu.SemaphoreType.DMA,
          pltpu.SemaphoreType.DMA,
          pltpu.SemaphoreType.DMA,
          pltpu.SemaphoreType.DMA,
      ],
  )(src, dst2, vals, x, zeros)

  part = part.reshape(NC, n, d_in)

  bt = 400
  grid = n // bt
  out = pl.pallas_call(
      _tc_finish_body,
      out_shape=jax.ShapeDtypeStruct((n, d_out), jnp.float32),
      grid=(grid,),
      in_specs=[
          pl.BlockSpec(memory_space=pltpu.SMEM),
          pl.BlockSpec((NC, bt, d_in), lambda i: (0, i, 0)),
          pl.BlockSpec((bt, d_in), lambda i: (i, 0)),
          pl.BlockSpec((d_in, d_out), lambda i: (0, 0)),
      ],
      out_specs=pl.BlockSpec((bt, d_out), lambda i: (i, 0)),
  )(alpha.reshape(1), part, x, W)
  return out


# speedup vs baseline: 1.5235x; 1.5235x over previous
import functools

import jax
import jax.numpy as jnp
from jax import lax
from jax.experimental import pallas as pl
from jax.experimental.pallas import tpu as pltpu
from jax.experimental.pallas import tpu_sc as plsc

NC = 2
NS = 16
L = 16
CHUNK = 128
PAGE = 16
PAGES_PER_TILE = 5


def _scale_rows(rows_ref, vals_ref, base, d):
  def scale_group(g, carry):
    vv = vals_ref[pl.ds(base + g * L, L)]
    for j in range(L):
      vj = lax.gather(
          vv, jnp.full((L, 1), j, jnp.int32),
          lax.GatherDimensionNumbers(offset_dims=(),
                                     collapsed_slice_dims=(0,),
                                     start_index_map=(0,)),
          slice_sizes=(1,),
          mode=lax.GatherScatterMode.PROMISE_IN_BOUNDS)
      for k in range(d // L):
        sl = (g * L + j, pl.ds(k * L, L))
        rows_ref[sl] = rows_ref[sl] * vj
    return carry

  lax.fori_loop(0, CHUNK // L, scale_group, 0, unroll=False)


def _sc_spmm_body(src_hbm, dst_hbm, vals_hbm, x_hbm, zeros_hbm, part_hbm,
                  acc_sh, src_v, dst_v, vals_v, rows0, rows1,
                  g0, g1, w0, w1):
  n, d = x_hbm.shape
  zrows = (n // NS) // 8 * 8
  rem = n - NS * zrows
  c = lax.axis_index("c")
  s = lax.axis_index("s")

  pltpu.sync_copy(zeros_hbm.at[pl.ds(s * zrows, zrows)],
                  acc_sh.at[pl.ds(s * zrows, zrows)])
  if rem:
    @pl.when(s == 0)
    def _():
      pltpu.sync_copy(zeros_hbm.at[pl.ds(NS * zrows, rem)],
                      acc_sh.at[pl.ds(NS * zrows, rem)])
  plsc.subcore_barrier()

  def gather_rows(chunk, rows_ref, sem):
    idx = src_v.at[pl.ds(chunk * CHUNK, CHUNK)]
    return pltpu.async_copy(x_hbm.at[idx], rows_ref, sem)

  def wait_gather(rows_ref, sem):
    idx = src_v.at[pl.ds(0, CHUNK)]
    pltpu.make_async_copy(x_hbm.at[idx], rows_ref, sem).wait()

  def scatter_rows(chunk, rows_ref, sem):
    return pltpu.async_copy(rows_ref, acc_sh.at[dst_v.at[chunk]], sem,
                            add=True)

  def wait_scatter(rows_ref, sem):
    pltpu.make_async_copy(rows_ref, acc_sh.at[dst_v.at[0]], sem).wait()

  def do_page(page):
    pltpu.sync_copy(src_hbm.at[pl.ds(page * PAGE * CHUNK, PAGE * CHUNK)],
                    src_v)
    pltpu.sync_copy(vals_hbm.at[pl.ds(page * PAGE * CHUNK, PAGE * CHUNK)],
                    vals_v)
    pltpu.sync_copy(dst_hbm.at[pl.ds(page * PAGE, PAGE)], dst_v)

    gather_rows(0, rows0, g0)
    gather_rows(1, rows1, g1)
    last = PAGE - 1

    def pair_body(i, carry2):
      c0 = 2 * i
      c1 = c0 + 1
      wait_gather(rows0, g0)
      _scale_rows(rows0, vals_v, c0 * CHUNK, d)
      scatter_rows(c0, rows0, w0)
      wait_scatter(rows0, w0)
      gather_rows(jnp.minimum(c0 + 2, last), rows0, g0)
      wait_gather(rows1, g1)
      _scale_rows(rows1, vals_v, c1 * CHUNK, d)
      scatter_rows(c1, rows1, w1)
      wait_scatter(rows1, w1)
      gather_rows(jnp.minimum(c1 + 2, last), rows1, g1)
      return carry2

    lax.fori_loop(0, PAGE // 2, pair_body, 0, unroll=False)
    wait_gather(rows0, g0)
    wait_gather(rows1, g1)

  def page_body(p, carry):
    do_page((c * NS + s) * PAGES_PER_TILE + p)
    return carry

  lax.fori_loop(0, PAGES_PER_TILE, page_body, 0, unroll=False)
  plsc.subcore_barrier()

  pltpu.sync_copy(acc_sh.at[pl.ds(s * zrows, zrows)],
                  part_hbm.at[pl.ds(c * n + s * zrows, zrows)])
  if rem:
    @pl.when(s == 0)
    def _():
      pltpu.sync_copy(acc_sh.at[pl.ds(NS * zrows, rem)],
                      part_hbm.at[pl.ds(c * n + NS * zrows, rem)])


def _tc_finish_body(a_ref, p_ref, x_ref, w_ref, o_ref):
  a = a_ref[0]
  blended = (1.0 - a) * (p_ref[0] + p_ref[1]) + a * x_ref[...]
  o_ref[...] = jnp.dot(blended, w_ref[...], preferred_element_type=jnp.float32)


def kernel(edge_index, adj_vals, x, alpha, W):
  n, d_in = x.shape
  d_out = W.shape[1]
  e = adj_vals.shape[0]

  dst = edge_index[0]
  src = edge_index[1]
  e_pad = NC * NS * PAGES_PER_TILE * PAGE * CHUNK
  assert e_pad >= e, "page split must cover all edges"
  if e_pad != e:
    pad = e_pad - e
    spread = jnp.arange(pad, dtype=jnp.int32) % n
    src = jnp.concatenate([src, spread])
    dst = jnp.concatenate([dst, spread])
    vals = jnp.concatenate([adj_vals, jnp.zeros((pad,), adj_vals.dtype)])
  else:
    vals = adj_vals
  dst2 = dst.reshape(e_pad // CHUNK, CHUNK)
  zeros = jnp.zeros((n, d_in), jnp.float32)

  mesh = plsc.VectorSubcoreMesh(core_axis_name="c", subcore_axis_name="s")
  part = pl.kernel(
      _sc_spmm_body,
      out_type=jax.ShapeDtypeStruct((NC * n, d_in), jnp.float32),
      mesh=mesh,
      scratch_types=[
          pltpu.VMEM_SHARED((n, d_in), jnp.float32),
          pltpu.VMEM((PAGE * CHUNK,), jnp.int32),
          pltpu.VMEM((PAGE, CHUNK), jnp.int32),
          pltpu.VMEM((PAGE * CHUNK,), jnp.float32),
          pltpu.VMEM((CHUNK, d_in), jnp.float32),
          pltpu.VMEM((CHUNK, d_in), jnp.float32),
          pltpu.SemaphoreType.DMA,
          pltpu.SemaphoreType.DMA,
          pltpu.SemaphoreType.DMA,
          pltpu.SemaphoreType.DMA,
      ],
  )(src, dst2, vals, x, zeros)

  part = part.reshape(NC, n, d_in)

  bt = 1000
  grid = n // bt
  out = pl.pallas_call(
      _tc_finish_body,
      out_shape=jax.ShapeDtypeStruct((n, d_out), jnp.float32),
      grid=(grid,),
      in_specs=[
          pl.BlockSpec(memory_space=pltpu.SMEM),
          pl.BlockSpec((NC, bt, d_in), lambda i: (0, i, 0)),
          pl.BlockSpec((bt, d_in), lambda i: (i, 0)),
          pl.BlockSpec((d_in, d_out), lambda i: (0, 0)),
      ],
      out_specs=pl.BlockSpec((bt, d_out), lambda i: (i, 0)),
  )(alpha.reshape(1), part, x, W)
  return out
